# baseline (speedup 1.0000x reference)
import functools

import jax
import jax.numpy as jnp
from jax.experimental import pallas as pl
from jax.experimental.pallas import tpu as pltpu

_BE = 3200
_EPS = 1e-5


def _edge_tc_body(first, has_u, last, *refs):
    i = 0
    if first:
        dist_ref = refs[i]; i += 1
        wedge_ref = refs[i]; i += 1
        bedge_ref = refs[i]; i += 1
    else:
        wprev_ref = refs[i]; i += 1
    if has_u:
        u_ref = refs[i]; i += 1
        a_ref = refs[i]; i += 1
        b_ref = refs[i]; i += 1
    if not last:
        we_ref = refs[i]; i += 1
        be_ref = refs[i]; i += 1
    wout_ref = refs[i]; i += 1
    if not last:
        tout_ref = refs[i]; i += 1

    if first:
        w_prev = dist_ref[...] * wedge_ref[...] + bedge_ref[...]
    else:
        w_prev = wprev_ref[...]
    if has_u:
        z = u_ref[...] * a_ref[...] + b_ref[...]
        w = w_prev + z * jax.nn.sigmoid(z)
    else:
        w = w_prev
    wout_ref[...] = w
    if not last:
        tout_ref[...] = (
            jnp.dot(w, we_ref[...], preferred_element_type=jnp.float32)
            + be_ref[...]
        )


def _edge_tc(first, has_u, last, *args, E, D):
    grid = (E // _BE,)
    row_spec = pl.BlockSpec((_BE, D), lambda i: (i, 0))
    dist_spec = pl.BlockSpec((_BE, 1), lambda i: (i, 0))
    vec_spec = pl.BlockSpec((1, D), lambda i: (0, 0))
    mat_spec = pl.BlockSpec((D, D), lambda i: (0, 0))
    in_specs = []
    if first:
        in_specs += [dist_spec, vec_spec, vec_spec]
    else:
        in_specs += [row_spec]
    if has_u:
        in_specs += [row_spec, vec_spec, vec_spec]
    if not last:
        in_specs += [mat_spec, vec_spec]
    out_specs = [row_spec] + ([] if last else [row_spec])
    out_shape = [jax.ShapeDtypeStruct((E, D), jnp.float32)]
    if not last:
        out_shape.append(jax.ShapeDtypeStruct((E, D), jnp.float32))
    out = pl.pallas_call(
        functools.partial(_edge_tc_body, first, has_u, last),
        grid=grid,
        in_specs=in_specs,
        out_specs=out_specs,
        out_shape=out_shape,
    )(*args)
    return out


def _node_body(nparts, lastlayer, *refs):
    i = 0
    x_ref = refs[i]; i += 1
    x1_ref = refs[i]; i += 1
    aggp_ref = refs[i]; i += 1
    counts_ref = refs[i]; i += 1
    gamma_ref = refs[i]; i += 1
    beta_ref = refs[i]; i += 1
    if not lastlayer:
        wv_ref = refs[i]; i += 1
        bv_ref = refs[i]; i += 1
    xout_ref = refs[i]; i += 1
    if not lastlayer:
        x1o_ref = refs[i]; i += 1
        x2o_ref = refs[i]; i += 1
        x3o_ref = refs[i]; i += 1
        x4o_ref = refs[i]; i += 1

    agg = aggp_ref[0]
    for p in range(1, nparts):
        agg = agg + aggp_ref[p]
    agg = agg / counts_ref[...]
    pre = x1_ref[...] + agg
    mean = jnp.mean(pre, axis=0, keepdims=True)
    var = jnp.mean((pre - mean) ** 2, axis=0, keepdims=True)
    xn = gamma_ref[...] * (pre - mean) / jnp.sqrt(var + _EPS) + beta_ref[...]
    x = x_ref[...] + xn * jax.nn.sigmoid(xn)
    xout_ref[...] = x
    if not lastlayer:
        outs = (x1o_ref, x2o_ref, x3o_ref, x4o_ref)
        for k in range(4):
            outs[k][...] = (
                jnp.dot(x, wv_ref[k], preferred_element_type=jnp.float32)
                + bv_ref[0, k][None, :]
            )


def _node_tc(x, x1, aggp, counts, gamma, beta, wv, bv, lastlayer, N, D):
    nparts = aggp.shape[0]
    args = [x, x1, aggp, counts, gamma.reshape(1, D), beta.reshape(1, D)]
    in_specs = [
        pl.BlockSpec((N, D), lambda: (0, 0)),
        pl.BlockSpec((N, D), lambda: (0, 0)),
        pl.BlockSpec((nparts, N, D), lambda: (0, 0, 0)),
        pl.BlockSpec((N, 1), lambda: (0, 0)),
        pl.BlockSpec((1, D), lambda: (0, 0)),
        pl.BlockSpec((1, D), lambda: (0, 0)),
    ]
    out_shape = [jax.ShapeDtypeStruct((N, D), jnp.float32)]
    if not lastlayer:
        args += [wv, bv.reshape(1, 4, D)]
        in_specs += [
            pl.BlockSpec((4, D, D), lambda: (0, 0, 0)),
            pl.BlockSpec((1, 4, D), lambda: (0, 0, 0)),
        ]
        out_shape += [jax.ShapeDtypeStruct((N, D), jnp.float32)] * 4
    out_specs = [pl.BlockSpec((N, D), lambda: (0, 0))] * len(out_shape)
    return pl.pallas_call(
        functools.partial(_node_body, nparts, lastlayer),
        in_specs=in_specs,
        out_specs=out_specs,
        out_shape=out_shape,
    )(*args)


def _init_body(locs_ref, wi_ref, bi_ref, wv_ref, bv_ref,
               ne_ref, x1o, x2o, x3o, x4o):
    ne = (
        jnp.dot(locs_ref[...], wi_ref[...], preferred_element_type=jnp.float32)
        + bi_ref[...]
    )
    ne_ref[...] = ne
    outs = (x1o, x2o, x3o, x4o)
    for k in range(4):
        outs[k][...] = (
            jnp.dot(ne, wv_ref[k], preferred_element_type=jnp.float32)
            + bv_ref[0, k][None, :]
        )


def _init_tc(locs, W_init, b_init, wv0, bv0, N, D):
    return pl.pallas_call(
        _init_body,
        in_specs=[
            pl.BlockSpec((N, 2), lambda: (0, 0)),
            pl.BlockSpec((2, D), lambda: (0, 0)),
            pl.BlockSpec((1, D), lambda: (0, 0)),
            pl.BlockSpec((4, D, D), lambda: (0, 0, 0)),
            pl.BlockSpec((1, 4, D), lambda: (0, 0, 0)),
        ],
        out_specs=[pl.BlockSpec((N, D), lambda: (0, 0))] * 5,
        out_shape=[jax.ShapeDtypeStruct((N, D), jnp.float32)] * 5,
    )(locs, W_init, b_init.reshape(1, D), wv0, bv0.reshape(1, 4, D))


def _bn_affine(s, ss, count, gamma, beta):
    mean = s / count
    var = ss / count - mean * mean
    inv = gamma / jnp.sqrt(var + _EPS)
    return inv, beta - mean * inv


def kernel(locs, edge_index, W_init, b_init, W_edge, b_edge, Wv, bv, We, be,
           gamma_v, beta_v, gamma_e, beta_e):
    N, D = locs.shape[0], W_init.shape[1]
    E = edge_index.shape[1]
    L = Wv.shape[0]
    src = edge_index[0]
    dst = edge_index[1]

    dl = locs[src] - locs[dst]
    dist = jnp.sqrt(dl[:, 0] ** 2 + dl[:, 1] ** 2 + 1e-12)
    ones = jnp.ones((E,), jnp.float32)
    counts = jnp.maximum(
        jax.ops.segment_sum(ones, src, num_segments=N), 1.0
    ).reshape(N, 1)

    dist2 = dist.reshape(E, 1)
    wedge = W_edge.reshape(1, D)
    bedge = b_edge.reshape(1, D)

    node_embed, x1, x2, x3, x4 = _init_tc(
        locs, W_init, b_init, Wv[0], bv[0], N, D)
    x = node_embed

    w_prev = None
    u_prev = None
    for l in range(L):
        if l == 0:
            q = (W_edge @ We[0]).reshape(1, D)
            r = (b_edge @ We[0] + be[0]).reshape(1, D)
            t = dist2 * q + r
            w_cur = dist2 * wedge + bedge
        elif l == 1:
            A, B = _bn_affine(su, ssu, float(E), gamma_e[l - 1], beta_e[l - 1])
            w_cur, t = _edge_tc(
                True, True, False,
                dist2, wedge, bedge, u_prev, A.reshape(1, D), B.reshape(1, D),
                We[l], be[l].reshape(1, D), E=E, D=D)
        else:
            A, B = _bn_affine(su, ssu, float(E), gamma_e[l - 1], beta_e[l - 1])
            w_cur, t = _edge_tc(
                False, True, False,
                w_prev, u_prev, A.reshape(1, D), B.reshape(1, D),
                We[l], be[l].reshape(1, D), E=E, D=D)

        sig = jax.nn.sigmoid(w_cur)
        msgs = sig * x2[dst]
        aggp = jax.ops.segment_sum(msgs, src, num_segments=N)[None]
        u = t + x3[src] + x4[dst]
        su = jnp.sum(u, axis=0)
        ssu = jnp.sum(u * u, axis=0)

        lastlayer = l == L - 1
        if lastlayer:
            outs = _node_tc(x, x1, aggp, counts, gamma_v[l], beta_v[l],
                            None, None, True, N, D)
            x = outs[0]
        else:
            x, x1, x2, x3, x4 = _node_tc(
                x, x1, aggp, counts, gamma_v[l], beta_v[l],
                Wv[l + 1], bv[l + 1], False, N, D)
        w_prev = w_cur
        u_prev = u

    A, B = _bn_affine(su, ssu, float(E), gamma_e[L - 1], beta_e[L - 1])
    (w_final,) = _edge_tc(
        False, True, True,
        w_prev, u_prev, A.reshape(1, D), B.reshape(1, D), E=E, D=D)
    return (x, w_final, node_embed)
